# baseline (speedup 1.0000x reference)
import functools

import jax
import jax.numpy as jnp
from jax import lax
from jax.experimental import pallas as pl
from jax.experimental.pallas import tpu as pltpu
from jax.experimental.pallas import tpu_sc as plsc

_NC = 2
_NS = 16
_B = 80


@functools.lru_cache(maxsize=None)
def _build_norm(E, N, R):
    ROWS = E // _B
    RPT = ROWS // _NS
    CH = 5
    RPC = RPT // CH
    NR = N * R
    WPT = NR // _NS
    mesh = plsc.VectorSubcoreMesh(core_axis_name="c", subcore_axis_name="s",
                                  num_cores=_NC, num_subcores=_NS)

    def body(dst_h, rid_h, src_h, norm_h, eidx_h,
             counts_sp, cvm, dstv, ridv, srcv, idxv, onesv, normv, eidxv, zb):
        c = lax.axis_index("c")
        s = lax.axis_index("s")
        z16 = jnp.zeros((16,), jnp.float32)
        for i in range(1024 // 16):
            zb[pl.ds(16 * i, 16)] = z16
        one16 = jnp.full((16,), 1.0, jnp.float32)
        for i in range(_B // 16):
            onesv[pl.ds(16 * i, 16)] = one16

        def zloop(m, carry):
            pltpu.sync_copy(zb.at[pl.ds(0, 1000)],
                            counts_sp.at[pl.ds(s * WPT + m * 1000, 1000)])
            return carry
        lax.fori_loop(0, WPT // 1000, zloop, 0)
        plsc.subcore_barrier()

        def p1(ch, carry):
            rb = s * RPT + ch * RPC
            pltpu.sync_copy(dst_h.at[pl.ds(rb, RPC)], dstv)
            pltpu.sync_copy(rid_h.at[pl.ds(rb, RPC)], ridv)

            def rows_(j, cc):
                for k in range(_B // 16):
                    sl = pl.ds(16 * k, 16)
                    idxv[j, sl] = dstv[j, sl] * R + ridv[j, sl]
                return cc
            lax.fori_loop(0, RPC, rows_, 0)

            def scat(j, cc):
                pltpu.sync_copy(onesv, counts_sp.at[idxv.at[j]], add=True)
                return cc
            lax.fori_loop(0, RPC, scat, 0)
            return carry
        lax.fori_loop(0, CH, p1, 0)
        plsc.subcore_barrier()

        pltpu.sync_copy(counts_sp, cvm)

        def p2(ch, carry):
            rb = s * RPT + ch * RPC
            pltpu.sync_copy(dst_h.at[pl.ds(rb, RPC)], dstv)
            pltpu.sync_copy(rid_h.at[pl.ds(rb, RPC)], ridv)
            pltpu.sync_copy(src_h.at[pl.ds(rb, RPC)], srcv)

            def rows_(j, cc):
                for k in range(_B // 16):
                    sl = pl.ds(16 * k, 16)
                    d16 = dstv[j, sl]
                    r16 = ridv[j, sl]
                    cnt = plsc.load_gather(cvm, [d16 * R + r16])
                    normv[j, sl] = 1.0 / jnp.maximum(cnt, 1.0)
                    eidxv[j, sl] = r16 * N + srcv[j, sl]
                return cc
            lax.fori_loop(0, RPC, rows_, 0)

            @pl.when(c == 0)
            def _():
                pltpu.sync_copy(normv, norm_h.at[pl.ds(rb, RPC)])
                pltpu.sync_copy(eidxv, eidx_h.at[pl.ds(rb, RPC)])
            return carry
        lax.fori_loop(0, CH, p2, 0)

    return pl.kernel(
        body,
        out_type=(jax.ShapeDtypeStruct((ROWS, _B), jnp.float32),
                  jax.ShapeDtypeStruct((ROWS, _B), jnp.int32)),
        mesh=mesh,
        compiler_params=pltpu.CompilerParams(use_tc_tiling_on_sc=False, needs_layout_passes=False),
        scratch_types=[
            pltpu.VMEM_SHARED((NR,), jnp.float32),
            pltpu.VMEM((NR,), jnp.float32),
            pltpu.VMEM((RPC, _B), jnp.int32),
            pltpu.VMEM((RPC, _B), jnp.int32),
            pltpu.VMEM((RPC, _B), jnp.int32),
            pltpu.VMEM((RPC, _B), jnp.int32),
            pltpu.VMEM((_B,), jnp.float32),
            pltpu.VMEM((RPC, _B), jnp.float32),
            pltpu.VMEM((RPC, _B), jnp.int32),
            pltpu.VMEM((1024,), jnp.float32),
        ],
    )


@functools.lru_cache(maxsize=None)
def _build_agg(E, N, HALF, RN):
    ROWS = E // _B
    RPT = ROWS // _NS
    CH = 5
    RPC = RPT // CH
    NPT = N // _NS
    ZR = 5
    mesh = plsc.VectorSubcoreMesh(core_axis_name="c", subcore_axis_name="s",
                                  num_cores=_NC, num_subcores=_NS)

    def body(tx_h, eidx_h, dst_h, norm_h, out_h,
             agg_sp, eidxv, dstv, normv, rb0, rb1, rb2, rf0, rf1, zb,
             g0, g1, g2, sS0, sS1):
        c = lax.axis_index("c")
        s = lax.axis_index("s")
        z16 = jnp.zeros((16,), jnp.float32)
        for i in range(ZR):
            for k in range(HALF // 16):
                zb[i, pl.ds(16 * k, 16)] = z16

        def zloop(m, carry):
            pltpu.sync_copy(zb, agg_sp.at[pl.ds(s * NPT + m * ZR, ZR)])
            return carry
        lax.fori_loop(0, NPT // ZR, zloop, 0)
        plsc.subcore_barrier()

        coff = c * RN
        ev2 = lax.iota(jnp.int32, 16) * 2

        def scale_rows(rowsb, rowf, j):
            def scale(g, ee):
                nv = normv[j, pl.ds(16 * g, 16)]
                for t in range(16):
                    sc = nv[t]
                    e = 16 * g + t
                    efull = jnp.full((16,), e, jnp.int32)
                    for k in range(HALF // 32):
                        w = plsc.bitcast(rowsb[e, pl.ds(32 * k, 32)],
                                         jnp.int32)
                        lo = plsc.bitcast(w << 16, jnp.float32) * sc
                        hi = plsc.bitcast(w & jnp.int32(-65536),
                                          jnp.float32) * sc
                        cols = ev2 + (32 * k)
                        plsc.store_scatter(rowf, [efull, cols], lo)
                        plsc.store_scatter(rowf, [efull, cols + 1], hi)
                return ee
            lax.fori_loop(0, _B // 16, scale, 0)

        def chunk(ch, carry):
            rbase = s * RPT + ch * RPC
            pltpu.sync_copy(eidx_h.at[pl.ds(rbase, RPC)], eidxv)
            pltpu.sync_copy(dst_h.at[pl.ds(rbase, RPC)], dstv)
            pltpu.sync_copy(norm_h.at[pl.ds(rbase, RPC)], normv)

            def mk(j, cc):
                for k in range(_B // 16):
                    sl = pl.ds(16 * k, 16)
                    eidxv[j, sl] = eidxv[j, sl] + coff
                return cc
            lax.fori_loop(0, RPC, mk, 0)

            gbufs = ((rb0, g0), (rb1, g1), (rb2, g2))
            fbufs = ((rf0, sS0), (rf1, sS1))
            pltpu.async_copy(tx_h.at[eidxv.at[0]], rb0, g0)
            pltpu.async_copy(tx_h.at[eidxv.at[1]], rb1, g1)

            def group(gp, cc):
                j0 = 6 * gp
                for t in range(6):
                    rbuf, gs = gbufs[t % 3]
                    nbuf, ngs = gbufs[(t + 2) % 3]
                    fbuf, ss = fbufs[t % 2]
                    j = j0 + t
                    jn = j + 2
                    pltpu.make_async_copy(tx_h.at[eidxv.at[j]], rbuf,
                                          gs).wait()

                    def fetch(jn=jn, nbuf=nbuf, ngs=ngs):
                        pltpu.async_copy(tx_h.at[eidxv.at[jn]], nbuf, ngs)

                    def drain(fbuf=fbuf, ss=ss):
                        pltpu.make_async_copy(
                            fbuf, agg_sp.at[dstv.at[0]], ss).wait()

                    if t < 5:
                        fetch()
                    else:
                        pl.when(jn < RPC)(fetch)
                    if t < 2:
                        pl.when(gp > 0)(drain)
                    else:
                        drain()
                    scale_rows(rbuf, fbuf, j)
                    pltpu.async_copy(fbuf, agg_sp.at[dstv.at[j]], ss,
                                     add=True)
                return cc
            lax.fori_loop(0, RPC // 6, group, 0)
            j_last = RPC - 1
            pltpu.make_async_copy(tx_h.at[eidxv.at[j_last]], rb0, g0).wait()
            pltpu.make_async_copy(rf0, agg_sp.at[dstv.at[0]], sS0).wait()
            scale_rows(rb0, rf0, j_last)
            pltpu.make_async_copy(rf1, agg_sp.at[dstv.at[0]], sS1).wait()
            pltpu.sync_copy(rf0, agg_sp.at[dstv.at[j_last]], add=True)
            return carry
        lax.fori_loop(0, CH, chunk, 0)

        plsc.subcore_barrier()
        pltpu.sync_copy(agg_sp.at[pl.ds(s * NPT, NPT)],
                        out_h.at[pl.ds(c * N + s * NPT, NPT)])

    return pl.kernel(
        body,
        out_type=jax.ShapeDtypeStruct((_NC * N, HALF), jnp.float32),
        mesh=mesh,
        compiler_params=pltpu.CompilerParams(use_tc_tiling_on_sc=False, needs_layout_passes=False),
        scratch_types=[
            pltpu.VMEM_SHARED((N, HALF), jnp.float32),
            pltpu.VMEM((RPC, _B), jnp.int32),
            pltpu.VMEM((RPC, _B), jnp.int32),
            pltpu.VMEM((RPC, _B), jnp.float32),
            pltpu.VMEM((_B, HALF), jnp.bfloat16),
            pltpu.VMEM((_B, HALF), jnp.bfloat16),
            pltpu.VMEM((_B, HALF), jnp.bfloat16),
            pltpu.VMEM((_B, HALF), jnp.float32),
            pltpu.VMEM((_B, HALF), jnp.float32),
            pltpu.VMEM((ZR, HALF), jnp.float32),
            pltpu.SemaphoreType.DMA,
            pltpu.SemaphoreType.DMA,
            pltpu.SemaphoreType.DMA,
            pltpu.SemaphoreType.DMA,
            pltpu.SemaphoreType.DMA,
        ],
    )


def _relmm(x, W):
    N, D = x.shape
    R, _, H = W.shape
    HALF = H // 2
    bn = 1000
    NB = N // bn

    def body(x_ref, w_ref, o_ref):
        o = jnp.dot(x_ref[...], w_ref[0], preferred_element_type=jnp.float32)
        o_ref[0] = o[:, :HALF].astype(jnp.bfloat16)
        o_ref[1] = o[:, HALF:].astype(jnp.bfloat16)

    return pl.pallas_call(
        body,
        grid=(NB, R),
        in_specs=[pl.BlockSpec((bn, D), lambda i, r: (i, 0)),
                  pl.BlockSpec((1, D, H), lambda i, r: (r, 0, 0))],
        out_specs=pl.BlockSpec((2, bn, HALF), lambda i, r: (0, r * NB + i, 0)),
        out_shape=jax.ShapeDtypeStruct((2, R * N, HALF), jnp.bfloat16),
    )(x, W).reshape(2 * R * N, HALF)


def _combine_mm(agg2, x, Wl, b2d, W2):
    N, D = x.shape
    H = Wl.shape[1]
    HALF = H // 2
    R = W2.shape[0]
    bn = 1000
    NB = N // bn

    def body(a_ref, x_ref, wl_ref, b_ref, w2_ref, h_ref, t_ref, hs):
        r = pl.program_id(1)

        @pl.when(r == 0)
        def _():
            h = (jnp.dot(x_ref[...], wl_ref[...],
                         preferred_element_type=jnp.float32) + b_ref[...])
            h = h + jnp.concatenate([a_ref[0], a_ref[1]], axis=-1)
            hs[...] = h
            h_ref[...] = h

        t = jnp.dot(hs[...], w2_ref[0], preferred_element_type=jnp.float32)
        t_ref[0] = t[:, :HALF].astype(jnp.bfloat16)
        t_ref[1] = t[:, HALF:].astype(jnp.bfloat16)

    h, tx2 = pl.pallas_call(
        body,
        grid=(NB, R),
        in_specs=[pl.BlockSpec((2, bn, HALF), lambda i, r: (0, i, 0)),
                  pl.BlockSpec((bn, D), lambda i, r: (i, 0)),
                  pl.BlockSpec((D, H), lambda i, r: (0, 0)),
                  pl.BlockSpec((1, H), lambda i, r: (0, 0)),
                  pl.BlockSpec((1, H, H), lambda i, r: (r, 0, 0))],
        out_specs=[pl.BlockSpec((bn, H), lambda i, r: (i, 0)),
                   pl.BlockSpec((2, bn, HALF), lambda i, r: (0, r * NB + i, 0))],
        out_shape=[jax.ShapeDtypeStruct((N, H), jnp.float32),
                   jax.ShapeDtypeStruct((2, R * N, HALF), jnp.bfloat16)],
        scratch_shapes=[pltpu.VMEM((bn, H), jnp.float32)],
    )(agg2, x, Wl, b2d, W2)
    return h, tx2.reshape(2 * R * N, HALF)


def _combine(agg2, x, Wl, b2d):
    N, D = x.shape
    H = Wl.shape[1]
    HALF = H // 2
    bn = 1000

    def body(a_ref, x_ref, w_ref, b_ref, o_ref):
        o_ref[...] = (jnp.dot(x_ref[...], w_ref[...],
                              preferred_element_type=jnp.float32)
                      + b_ref[...])
        o_ref[:, 0:HALF] += a_ref[0]
        o_ref[:, HALF:H] += a_ref[1]

    return pl.pallas_call(
        body,
        grid=(N // bn,),
        in_specs=[pl.BlockSpec((2, bn, HALF), lambda i: (0, i, 0)),
                  pl.BlockSpec((bn, D), lambda i: (i, 0)),
                  pl.BlockSpec((D, H), lambda i: (0, 0)),
                  pl.BlockSpec((1, H), lambda i: (0, 0))],
        out_specs=pl.BlockSpec((bn, H), lambda i: (i, 0)),
        out_shape=jax.ShapeDtypeStruct((N, H), jnp.float32),
    )(agg2, x, Wl, b2d)


def kernel(node_feats, edge_index, rel_ids, W1, Wloop1, b1, W2, Wloop2, b2):
    N, D = node_feats.shape
    R, _, H = W1.shape
    E = edge_index.shape[1]
    HALF = H // 2

    src = edge_index[0]
    dst = edge_index[1]
    dst2 = dst.reshape(-1, _B)
    rid2 = rel_ids.reshape(-1, _B)
    src2 = src.reshape(-1, _B)
    norm2, eidx2 = _build_norm(E, N, R)(dst2, rid2, src2)

    agg_call = _build_agg(E, N, HALF, R * N)
    tx1 = _relmm(node_feats, W1)
    agg1 = agg_call(tx1, eidx2, dst2, norm2)
    h1, tx2 = _combine_mm(agg1.reshape(2, N, HALF), node_feats, Wloop1,
                          b1.reshape(1, H), W2)
    agg2 = agg_call(tx2, eidx2, dst2, norm2)
    h2 = _combine(agg2.reshape(2, N, HALF), h1, Wloop2, b2.reshape(1, H))
    return h2


# speedup vs baseline: 2.3557x; 2.3557x over previous
import functools

import jax
import jax.numpy as jnp
from jax import lax
from jax.experimental import pallas as pl
from jax.experimental.pallas import tpu as pltpu
from jax.experimental.pallas import tpu_sc as plsc

_NC = 2
_NS = 16
_B = 80


@functools.lru_cache(maxsize=None)
def _build_norm(E, N, R):
    ROWS = E // _B
    RPT = ROWS // _NS
    CH = 5
    RPC = RPT // CH
    NR = N * R
    WPT = NR // _NS
    mesh = plsc.VectorSubcoreMesh(core_axis_name="c", subcore_axis_name="s",
                                  num_cores=_NC, num_subcores=_NS)

    def body(dst_h, rid_h, src_h, norm_h, eidx_h,
             counts_sp, cvm, dstv, ridv, srcv, idxv, onesv, normv, eidxv, zb):
        c = lax.axis_index("c")
        s = lax.axis_index("s")
        z16 = jnp.zeros((16,), jnp.float32)
        for i in range(1024 // 16):
            zb[pl.ds(16 * i, 16)] = z16
        one16 = jnp.full((16,), 1.0, jnp.float32)
        for i in range(_B // 16):
            onesv[pl.ds(16 * i, 16)] = one16

        def zloop(m, carry):
            pltpu.sync_copy(zb.at[pl.ds(0, 1000)],
                            counts_sp.at[pl.ds(s * WPT + m * 1000, 1000)])
            return carry
        lax.fori_loop(0, WPT // 1000, zloop, 0)
        plsc.subcore_barrier()

        def p1(ch, carry):
            rb = s * RPT + ch * RPC
            pltpu.sync_copy(dst_h.at[pl.ds(rb, RPC)], dstv)
            pltpu.sync_copy(rid_h.at[pl.ds(rb, RPC)], ridv)

            def rows_(j, cc):
                for k in range(_B // 16):
                    sl = pl.ds(16 * k, 16)
                    idxv[j, sl] = dstv[j, sl] * R + ridv[j, sl]
                return cc
            lax.fori_loop(0, RPC, rows_, 0)

            def scat(j, cc):
                pltpu.sync_copy(onesv, counts_sp.at[idxv.at[j]], add=True)
                return cc
            lax.fori_loop(0, RPC, scat, 0)
            return carry
        lax.fori_loop(0, CH, p1, 0)
        plsc.subcore_barrier()

        pltpu.sync_copy(counts_sp, cvm)

        def p2(ch, carry):
            rb = s * RPT + ch * RPC
            pltpu.sync_copy(dst_h.at[pl.ds(rb, RPC)], dstv)
            pltpu.sync_copy(rid_h.at[pl.ds(rb, RPC)], ridv)
            pltpu.sync_copy(src_h.at[pl.ds(rb, RPC)], srcv)

            def rows_(j, cc):
                for k in range(_B // 16):
                    sl = pl.ds(16 * k, 16)
                    d16 = dstv[j, sl]
                    r16 = ridv[j, sl]
                    cnt = plsc.load_gather(cvm, [d16 * R + r16])
                    normv[j, sl] = 1.0 / jnp.maximum(cnt, 1.0)
                    eidxv[j, sl] = r16 * N + srcv[j, sl]
                return cc
            lax.fori_loop(0, RPC, rows_, 0)

            @pl.when(c == 0)
            def _():
                pltpu.sync_copy(normv, norm_h.at[pl.ds(rb, RPC)])
                pltpu.sync_copy(eidxv, eidx_h.at[pl.ds(rb, RPC)])
            return carry
        lax.fori_loop(0, CH, p2, 0)

    return pl.kernel(
        body,
        out_type=(jax.ShapeDtypeStruct((ROWS, _B), jnp.float32),
                  jax.ShapeDtypeStruct((ROWS, _B), jnp.int32)),
        mesh=mesh,
        compiler_params=pltpu.CompilerParams(use_tc_tiling_on_sc=False, needs_layout_passes=False),
        scratch_types=[
            pltpu.VMEM_SHARED((NR,), jnp.float32),
            pltpu.VMEM((NR,), jnp.float32),
            pltpu.VMEM((RPC, _B), jnp.int32),
            pltpu.VMEM((RPC, _B), jnp.int32),
            pltpu.VMEM((RPC, _B), jnp.int32),
            pltpu.VMEM((RPC, _B), jnp.int32),
            pltpu.VMEM((_B,), jnp.float32),
            pltpu.VMEM((RPC, _B), jnp.float32),
            pltpu.VMEM((RPC, _B), jnp.int32),
            pltpu.VMEM((1024,), jnp.float32),
        ],
    )


@functools.lru_cache(maxsize=None)
def _build_agg(E, N, HALF, RN):
    ROWS = E // _B
    RPT = ROWS // _NS
    CH = 5
    RPC = RPT // CH
    NPT = N // _NS
    ZR = 25
    mesh = plsc.VectorSubcoreMesh(core_axis_name="c", subcore_axis_name="s",
                                  num_cores=_NC, num_subcores=_NS)

    def body(tx_h, eidx_h, dst_h, norm_h, out_h,
             agg_sp, eidxv, dstv, normv, rows0, rows1, rows2, rows3, zb,
             g0, g1, g2, g3, s0, s1, s2, s3):
        c = lax.axis_index("c")
        s = lax.axis_index("s")
        z16 = jnp.zeros((16,), jnp.float32)
        for i in range(ZR):
            for k in range(HALF // 16):
                zb[i, pl.ds(16 * k, 16)] = z16

        def zloop(m, carry):
            pltpu.sync_copy(zb, agg_sp.at[pl.ds(s * NPT + m * ZR, ZR)])
            return carry
        lax.fori_loop(0, NPT // ZR, zloop, 0)
        plsc.subcore_barrier()

        coff = c * RN

        def scale_rows(rows, j):
            def scale(g, ee):
                nv = normv[j, pl.ds(16 * g, 16)]
                for t in range(16):
                    sc = nv[t]
                    e = 16 * g + t
                    for k in range(HALF // 16):
                        sl = pl.ds(16 * k, 16)
                        rows[e, sl] = rows[e, sl] * sc
                return ee
            lax.fori_loop(0, _B // 16, scale, 0)

        def chunk(ch, carry):
            rb = s * RPT + ch * RPC
            pltpu.sync_copy(eidx_h.at[pl.ds(rb, RPC)], eidxv)
            pltpu.sync_copy(dst_h.at[pl.ds(rb, RPC)], dstv)
            pltpu.sync_copy(norm_h.at[pl.ds(rb, RPC)], normv)

            def mk(j, cc):
                for k in range(_B // 16):
                    sl = pl.ds(16 * k, 16)
                    eidxv[j, sl] = eidxv[j, sl] + coff
                return cc
            lax.fori_loop(0, RPC, mk, 0)

            bufs = ((rows0, g0, s0), (rows1, g1, s1),
                    (rows2, g2, s2), (rows3, g3, s3))
            pltpu.async_copy(tx_h.at[eidxv.at[0]], rows0, g0)
            pltpu.async_copy(tx_h.at[eidxv.at[1]], rows1, g1)
            pltpu.async_copy(tx_h.at[eidxv.at[2]], rows2, g2)

            def group(gp, cc):
                j0 = 4 * gp
                for t in range(4):
                    rw, gs, ss = bufs[t]
                    nrw, ngs, nss = bufs[(t + 3) % 4]
                    j = j0 + t
                    jn = j + 3
                    pltpu.make_async_copy(tx_h.at[eidxv.at[j]], rw, gs).wait()
                    scale_rows(rw, j)
                    pltpu.async_copy(rw, agg_sp.at[dstv.at[j]], ss, add=True)

                    def drain(nrw=nrw, nss=nss):
                        pltpu.make_async_copy(
                            nrw, agg_sp.at[dstv.at[0]], nss).wait()

                    def fetch(jn=jn, nrw=nrw, ngs=ngs):
                        pltpu.async_copy(tx_h.at[eidxv.at[jn]], nrw, ngs)

                    if t == 0:
                        pl.when(gp > 0)(drain)
                        fetch()
                    elif t == 1:
                        drain()
                        fetch()
                    else:
                        drain()
                        pl.when(jn < RPC)(fetch)
                return cc
            lax.fori_loop(0, RPC // 4, group, 0)
            j_last = RPC - 1
            pltpu.make_async_copy(tx_h.at[eidxv.at[j_last]], rows0, g0).wait()
            scale_rows(rows0, j_last)
            pltpu.make_async_copy(rows3, agg_sp.at[dstv.at[0]], s3).wait()
            pltpu.sync_copy(rows0, agg_sp.at[dstv.at[j_last]], add=True)
            return carry
        lax.fori_loop(0, CH, chunk, 0)

        plsc.subcore_barrier()
        pltpu.sync_copy(agg_sp.at[pl.ds(s * NPT, NPT)],
                        out_h.at[pl.ds(c * N + s * NPT, NPT)])

    return pl.kernel(
        body,
        out_type=jax.ShapeDtypeStruct((_NC * N, HALF), jnp.float32),
        mesh=mesh,
        compiler_params=pltpu.CompilerParams(use_tc_tiling_on_sc=False, needs_layout_passes=False),
        scratch_types=[
            pltpu.VMEM_SHARED((N, HALF), jnp.float32),
            pltpu.VMEM((RPC, _B), jnp.int32),
            pltpu.VMEM((RPC, _B), jnp.int32),
            pltpu.VMEM((RPC, _B), jnp.float32),
            pltpu.VMEM((_B, HALF), jnp.float32),
            pltpu.VMEM((_B, HALF), jnp.float32),
            pltpu.VMEM((_B, HALF), jnp.float32),
            pltpu.VMEM((_B, HALF), jnp.float32),
            pltpu.VMEM((ZR, HALF), jnp.float32),
            pltpu.SemaphoreType.DMA,
            pltpu.SemaphoreType.DMA,
            pltpu.SemaphoreType.DMA,
            pltpu.SemaphoreType.DMA,
            pltpu.SemaphoreType.DMA,
            pltpu.SemaphoreType.DMA,
            pltpu.SemaphoreType.DMA,
            pltpu.SemaphoreType.DMA,
        ],
    )


def _relmm(x, W):
    N, D = x.shape
    R, _, H = W.shape
    HALF = H // 2
    bn = 1000
    NB = N // bn

    def body(x_ref, w_ref, o_ref):
        o = jnp.dot(x_ref[...], w_ref[0], preferred_element_type=jnp.float32)
        o_ref[0] = o[:, :HALF]
        o_ref[1] = o[:, HALF:]

    return pl.pallas_call(
        body,
        grid=(NB, R),
        in_specs=[pl.BlockSpec((bn, D), lambda i, r: (i, 0)),
                  pl.BlockSpec((1, D, H), lambda i, r: (r, 0, 0))],
        out_specs=pl.BlockSpec((2, bn, HALF), lambda i, r: (0, r * NB + i, 0)),
        out_shape=jax.ShapeDtypeStruct((2, R * N, HALF), jnp.float32),
    )(x, W).reshape(2 * R * N, HALF)


def _combine_mm(agg2, x, Wl, b2d, W2):
    N, D = x.shape
    H = Wl.shape[1]
    HALF = H // 2
    R = W2.shape[0]
    bn = 1000
    NB = N // bn

    def body(a_ref, x_ref, wl_ref, b_ref, w2_ref, h_ref, t_ref, hs):
        r = pl.program_id(1)

        @pl.when(r == 0)
        def _():
            h = (jnp.dot(x_ref[...], wl_ref[...],
                         preferred_element_type=jnp.float32) + b_ref[...])
            h = h + jnp.concatenate([a_ref[0], a_ref[1]], axis=-1)
            hs[...] = h
            h_ref[...] = h

        t = jnp.dot(hs[...], w2_ref[0], preferred_element_type=jnp.float32)
        t_ref[0] = t[:, :HALF]
        t_ref[1] = t[:, HALF:]

    h, tx2 = pl.pallas_call(
        body,
        grid=(NB, R),
        in_specs=[pl.BlockSpec((2, bn, HALF), lambda i, r: (0, i, 0)),
                  pl.BlockSpec((bn, D), lambda i, r: (i, 0)),
                  pl.BlockSpec((D, H), lambda i, r: (0, 0)),
                  pl.BlockSpec((1, H), lambda i, r: (0, 0)),
                  pl.BlockSpec((1, H, H), lambda i, r: (r, 0, 0))],
        out_specs=[pl.BlockSpec((bn, H), lambda i, r: (i, 0)),
                   pl.BlockSpec((2, bn, HALF), lambda i, r: (0, r * NB + i, 0))],
        out_shape=[jax.ShapeDtypeStruct((N, H), jnp.float32),
                   jax.ShapeDtypeStruct((2, R * N, HALF), jnp.float32)],
        scratch_shapes=[pltpu.VMEM((bn, H), jnp.float32)],
    )(agg2, x, Wl, b2d, W2)
    return h, tx2.reshape(2 * R * N, HALF)


def _combine(agg2, x, Wl, b2d):
    N, D = x.shape
    H = Wl.shape[1]
    HALF = H // 2
    bn = 1000

    def body(a_ref, x_ref, w_ref, b_ref, o_ref):
        o_ref[...] = (jnp.dot(x_ref[...], w_ref[...],
                              preferred_element_type=jnp.float32)
                      + b_ref[...])
        o_ref[:, 0:HALF] += a_ref[0]
        o_ref[:, HALF:H] += a_ref[1]

    return pl.pallas_call(
        body,
        grid=(N // bn,),
        in_specs=[pl.BlockSpec((2, bn, HALF), lambda i: (0, i, 0)),
                  pl.BlockSpec((bn, D), lambda i: (i, 0)),
                  pl.BlockSpec((D, H), lambda i: (0, 0)),
                  pl.BlockSpec((1, H), lambda i: (0, 0))],
        out_specs=pl.BlockSpec((bn, H), lambda i: (i, 0)),
        out_shape=jax.ShapeDtypeStruct((N, H), jnp.float32),
    )(agg2, x, Wl, b2d)


def kernel(node_feats, edge_index, rel_ids, W1, Wloop1, b1, W2, Wloop2, b2):
    N, D = node_feats.shape
    R, _, H = W1.shape
    E = edge_index.shape[1]
    HALF = H // 2

    src = edge_index[0]
    dst = edge_index[1]
    dst2 = dst.reshape(-1, _B)
    rid2 = rel_ids.reshape(-1, _B)
    src2 = src.reshape(-1, _B)
    norm2, eidx2 = _build_norm(E, N, R)(dst2, rid2, src2)

    agg_call = _build_agg(E, N, HALF, R * N)
    tx1 = _relmm(node_feats, W1)
    agg1 = agg_call(tx1, eidx2, dst2, norm2)
    h1, tx2 = _combine_mm(agg1.reshape(2, N, HALF), node_feats, Wloop1,
                          b1.reshape(1, H), W2)
    agg2 = agg_call(tx2, eidx2, dst2, norm2)
    h2 = _combine(agg2.reshape(2, N, HALF), h1, Wloop2, b2.reshape(1, H))
    return h2
